# Initial kernel scaffold; baseline (speedup 1.0000x reference)
import jax
import jax.numpy as jnp
from jax.experimental import pallas as pl


def kernel(x, edge_index, idx, edge_type, edge_weight, Wp, bp, bn_gamma, bn_beta, W_input0, b_input0, W_rel0, W_out0, b_out0, W_input1, b_input1, W_rel1, W_out1, b_out1):
    raise NotImplementedError("write your pallas kernel here")


# speedup vs baseline: 2.0510x; 2.0510x over previous
import jax
import jax.numpy as jnp
from jax.experimental import pallas as pl

N = 48758
E = 780128
D = 64
R = 4
B = 4096


def kernel(x, edge_index, idx, edge_type, edge_weight, Wp, bp, bn_gamma, bn_beta,
           W_input0, b_input0, W_rel0, W_out0, b_out0,
           W_input1, b_input1, W_rel1, W_out1, b_out1):
    row, col = edge_index[0], edge_index[1]
    z = x @ Wp.T + bp
    mu = jnp.mean(z, axis=0)
    var = jnp.var(z, axis=0)
    h = jax.nn.relu((z - mu) / jnp.sqrt(var + 1e-5) * bn_gamma + bn_beta)
    deg = jnp.zeros((N,), jnp.float32).at[col].add(1.0)
    dis = jnp.where(deg > 0, deg ** -0.5, 0.0)
    for (Wi, bi, Wr, Wo, bo) in ((W_input0, b_input0, W_rel0, W_out0, b_out0),
                                 (W_input1, b_input1, W_rel1, W_out1, b_out1)):
        xl = h @ Wi.T + bi
        xls = xl * dis[:, None]
        T = jnp.einsum('nd,rde->rne', xl, Wr).reshape(R * N, D)
        y = T[edge_type * N + col]
        e = jnp.exp(y)
        G = jnp.zeros((N, D), jnp.float32).at[row].add(xls[col]) * dis[:, None]
        denom = jnp.zeros((N, D), jnp.float32).at[row].add(e)
        num = jnp.zeros((N, D), jnp.float32).at[row].add(y * e)
        msg = num / (denom + 1e-16)
        h = (G + 0.1 * jax.nn.relu(msg)) @ Wo.T + bo
    return jax.nn.gelu(h)[idx]


# speedup vs baseline: 6.7134x; 3.2732x over previous
import functools
import jax
import jax.numpy as jnp
from jax import lax
from jax.experimental import pallas as pl
from jax.experimental.pallas import tpu as pltpu
from jax.experimental.pallas import tpu_sc as plsc

N = 48758
E = 780128
D = 64
R = 4
B = 4096

NC = 2
NS = 16
NW = NC * NS
L = 16

NBLK = 48
NPAD = NBLK * 1024
DUMMY = N
K = 512
NB = 48
EPW = K * NB
EPAD = NW * EPW
RPT = NPAD // NS

_MESH = plsc.VectorSubcoreMesh(core_axis_name="c", subcore_axis_name="s")

_f32 = jnp.float32
_i32 = jnp.int32


def _fill_const(ref, n_rows, value):
    v = jnp.full((L,), value, _f32)

    def body(i, _):
        ref[i] = v
        return 0

    lax.fori_loop(0, n_rows, body, 0)


def _deg_body(col_hbm, deg_out, acc, colv, ones2d):
    cid = lax.axis_index("c")
    sid = lax.axis_index("s")
    wid = sid * NC + cid
    base = wid * EPW
    r0 = sid * RPT

    _fill_const(ones2d, K, 0.0)
    for j in range(RPT // K):
        pltpu.sync_copy(ones2d, acc.at[pl.ds(r0 + j * K, K)])
    _fill_const(ones2d, K, 1.0)
    plsc.subcore_barrier()

    def batch(b, _):
        pltpu.sync_copy(col_hbm.at[pl.ds(base + b * K, K)], colv)
        pltpu.sync_copy(ones2d, acc.at[colv], add=True)
        return 0

    lax.fori_loop(0, NB, batch, 0)
    plsc.subcore_barrier()
    pltpu.sync_copy(acc.at[pl.ds(r0, RPT)], deg_out.at[cid, pl.ds(r0, RPT)])


_SC_PARAMS = pltpu.CompilerParams(use_tc_tiling_on_sc=False)

_deg_kernel = functools.partial(
    pl.kernel,
    out_type=jax.ShapeDtypeStruct((NC, NPAD, L), _f32),
    mesh=_MESH,
    compiler_params=_SC_PARAMS,
    scratch_types=[
        pltpu.VMEM_SHARED((NPAD, L), _f32),
        pltpu.VMEM((K,), _i32),
        pltpu.VMEM((K, L), _f32),
    ],
)(_deg_body)


def _scan_body(row_hbm, col_hbm, rel_hbm, x0, x1, x2, x3, t0, t1, t2, t3,
               out, acc1, acc2, rowv, colv, relv, gixv, ybuf, stg1, stg2):
    xs = (x0, x1, x2, x3)
    ts = (t0, t1, t2, t3)
    cid = lax.axis_index("c")
    sid = lax.axis_index("s")
    wid = sid * NC + cid
    base = wid * EPW
    r0 = sid * RPT

    def zero_accs():
        _fill_const(stg1, K, 0.0)
        for j in range(RPT // K):
            pltpu.sync_copy(stg1, acc1.at[pl.ds(r0 + j * K, K)])
            pltpu.sync_copy(stg1, acc2.at[pl.ds(r0 + j * K, K)])
        plsc.subcore_barrier()

    def flush(j1, j2):
        plsc.subcore_barrier()
        pltpu.sync_copy(acc1.at[pl.ds(r0, RPT)], out.at[j1, cid, pl.ds(r0, RPT)])
        pltpu.sync_copy(acc2.at[pl.ds(r0, RPT)], out.at[j2, cid, pl.ds(r0, RPT)])
        plsc.subcore_barrier()

    for p in range(2):
        zero_accs()

        def batch_a(b, _):
            pltpu.sync_copy(row_hbm.at[pl.ds(base + b * K, K)], rowv)
            pltpu.sync_copy(col_hbm.at[pl.ds(base + b * K, K)], colv)
            pltpu.sync_copy(xs[2 * p].at[colv], stg1)
            pltpu.sync_copy(xs[2 * p + 1].at[colv], stg2)
            pltpu.sync_copy(stg1, acc1.at[rowv], add=True)
            pltpu.sync_copy(stg2, acc2.at[rowv], add=True)
            return 0

        lax.fori_loop(0, NB, batch_a, 0)
        flush(2 * p, 2 * p + 1)

    for fc in range(4):
        zero_accs()

        def batch_b(b, _):
            pltpu.sync_copy(row_hbm.at[pl.ds(base + b * K, K)], rowv)
            pltpu.sync_copy(col_hbm.at[pl.ds(base + b * K, K)], colv)
            pltpu.sync_copy(rel_hbm.at[pl.ds(base + b * K, K)], relv)

            def gfill(i, _):
                s = pl.ds(i * L, L)
                gixv[s] = relv[s] * NPAD + colv[s]
                return 0

            lax.fori_loop(0, K // L, gfill, 0)
            pltpu.sync_copy(ts[fc].at[gixv], ybuf)

            def comp(k, _):
                y = ybuf[k]
                e = jnp.exp(y)
                stg1[k] = e
                stg2[k] = y * e
                return 0

            lax.fori_loop(0, K, comp, 0)
            pltpu.sync_copy(stg1, acc1.at[rowv], add=True)
            pltpu.sync_copy(stg2, acc2.at[rowv], add=True)
            return 0

        lax.fori_loop(0, NB, batch_b, 0)
        flush(4 + fc, 8 + fc)


_scan_kernel = functools.partial(
    pl.kernel,
    out_type=jax.ShapeDtypeStruct((12, NC, NPAD, L), _f32),
    mesh=_MESH,
    compiler_params=_SC_PARAMS,
    scratch_types=[
        pltpu.VMEM_SHARED((NPAD, L), _f32),
        pltpu.VMEM_SHARED((NPAD, L), _f32),
        pltpu.VMEM((K,), _i32),
        pltpu.VMEM((K,), _i32),
        pltpu.VMEM((K,), _i32),
        pltpu.VMEM((K,), _i32),
        pltpu.VMEM((K, L), _f32),
        pltpu.VMEM((K, L), _f32),
        pltpu.VMEM((K, L), _f32),
    ],
)(_scan_body)


def _take_body(h_hbm, idx_hbm, out_hbm, idxv, rows):
    cid = lax.axis_index("c")
    sid = lax.axis_index("s")
    wid = sid * NC + cid
    per = B // NW
    base = wid * per
    pltpu.sync_copy(idx_hbm.at[pl.ds(base, per)], idxv)
    pltpu.sync_copy(h_hbm.at[idxv], rows)
    pltpu.sync_copy(rows, out_hbm.at[pl.ds(base, per)])


_take_kernel = functools.partial(
    pl.kernel,
    out_type=jax.ShapeDtypeStruct((B, D), _f32),
    mesh=_MESH,
    compiler_params=_SC_PARAMS,
    scratch_types=[
        pltpu.VMEM((B // NW,), _i32),
        pltpu.VMEM((B // NW, D), _f32),
    ],
)(_take_body)


def _stats_body(x_ref, wp_ref, bp_ref, o_ref):
    i = pl.program_id(0)
    z = jnp.dot(x_ref[...], wp_ref[...].T, preferred_element_type=_f32) + bp_ref[...]
    s = jnp.sum(z, axis=0, keepdims=True)
    sq = jnp.sum(z * z, axis=0, keepdims=True)
    blk = jnp.concatenate([s, sq, jnp.zeros((6, D), _f32)], axis=0)

    @pl.when(i == 0)
    def _():
        o_ref[...] = blk

    @pl.when(i > 0)
    def _():
        o_ref[...] += blk


def _h_body(x_ref, wp_ref, bp_ref, g_ref, bt_ref, st_ref, o_ref):
    z = jnp.dot(x_ref[...], wp_ref[...].T, preferred_element_type=_f32) + bp_ref[...]
    npad_extra = float(NPAD - N)
    bp = bp_ref[...]
    ssum = st_ref[0:1, :] - npad_extra * bp
    ssq = st_ref[1:2, :] - npad_extra * bp * bp
    mu = ssum / float(N)
    var = ssq / float(N) - mu * mu
    hn = (z - mu) * lax.rsqrt(var + 1e-5) * g_ref[...] + bt_ref[...]
    o_ref[...] = jnp.maximum(hn, 0.0)


def _prep_body(h_ref, deg_ref, wi_ref, bi_ref, wr_ref,
               xo0, xo1, xo2, xo3, to0, to1, to2, to3):
    xos = (xo0, xo1, xo2, xo3)
    tos = (to0, to1, to2, to3)
    xl = jnp.dot(h_ref[...], wi_ref[...].T, preferred_element_type=_f32) + bi_ref[...]
    deg = deg_ref[0, :, 0:1] + deg_ref[1, :, 0:1]
    dis = jnp.where(deg > 0, lax.rsqrt(deg), 0.0)
    xsc = xl * dis
    for fc in range(4):
        xos[fc][...] = xsc[:, fc * L:(fc + 1) * L]
    for r in range(R):
        y = jnp.dot(xl, wr_ref[64 * r:64 * (r + 1), :], preferred_element_type=_f32)
        for fc in range(4):
            tos[fc][r] = y[:, fc * L:(fc + 1) * L]


def _finish_body(sc_ref, deg_ref, wo_ref, bo_ref, o_ref):
    deg = deg_ref[0, :, 0:1] + deg_ref[1, :, 0:1]
    dis = jnp.where(deg > 0, lax.rsqrt(deg), 0.0)
    g = jnp.concatenate(
        [sc_ref[f, 0] + sc_ref[f, 1] for f in range(4)], axis=1)
    den = jnp.concatenate(
        [sc_ref[4 + f, 0] + sc_ref[4 + f, 1] for f in range(4)], axis=1)
    num = jnp.concatenate(
        [sc_ref[8 + f, 0] + sc_ref[8 + f, 1] for f in range(4)], axis=1)
    msg = num / (den + 1e-16)
    t = g * dis + 0.1 * jnp.maximum(msg, 0.0)
    o_ref[...] = jnp.dot(t, wo_ref[...].T, preferred_element_type=_f32) + bo_ref[...]


def _gelu_body(x_ref, o_ref):
    o_ref[...] = jax.nn.gelu(x_ref[...])


def _row_spec():
    return pl.BlockSpec((1024, D), lambda i: (i, 0))


def _w_spec(shape):
    return pl.BlockSpec(shape, lambda i: tuple(0 for _ in shape))


def _deg_spec():
    return pl.BlockSpec((NC, 1024, L), lambda i: (0, i, 0))


def _tc_stats(xpad, Wp, bp2):
    return pl.pallas_call(
        _stats_body,
        grid=(NBLK,),
        in_specs=[_row_spec(), _w_spec((D, D)), _w_spec((1, D))],
        out_specs=pl.BlockSpec((8, D), lambda i: (0, 0)),
        out_shape=jax.ShapeDtypeStruct((8, D), _f32),
    )(xpad, Wp, bp2)


def _tc_h(xpad, Wp, bp2, g2, bt2, stats):
    return pl.pallas_call(
        _h_body,
        grid=(NBLK,),
        in_specs=[_row_spec(), _w_spec((D, D)), _w_spec((1, D)),
                  _w_spec((1, D)), _w_spec((1, D)), _w_spec((8, D))],
        out_specs=_row_spec(),
        out_shape=jax.ShapeDtypeStruct((NPAD, D), _f32),
    )(xpad, Wp, bp2, g2, bt2, stats)


def _tc_prep(h, deg, Wi, bi2, Wr):
    xspec = pl.BlockSpec((1024, L), lambda i: (i, 0))
    tspec = pl.BlockSpec((R, 1024, L), lambda i: (0, i, 0))
    outs = pl.pallas_call(
        _prep_body,
        grid=(NBLK,),
        in_specs=[_row_spec(), _deg_spec(), _w_spec((D, D)), _w_spec((1, D)),
                  _w_spec((R * D, D))],
        out_specs=[xspec] * 4 + [tspec] * 4,
        out_shape=([jax.ShapeDtypeStruct((NPAD, L), _f32)] * 4
                   + [jax.ShapeDtypeStruct((R, NPAD, L), _f32)] * 4),
    )(h, deg, Wi, bi2, Wr)
    return outs[:4], [t.reshape(R * NPAD, L) for t in outs[4:]]


def _tc_finish(sc, deg, Wo, bo2):
    return pl.pallas_call(
        _finish_body,
        grid=(NBLK,),
        in_specs=[pl.BlockSpec((12, NC, 1024, L), lambda i: (0, 0, i, 0)),
                  _deg_spec(), _w_spec((D, D)), _w_spec((1, D))],
        out_specs=_row_spec(),
        out_shape=jax.ShapeDtypeStruct((NPAD, D), _f32),
    )(sc, deg, Wo, bo2)


def _tc_gelu(x):
    return pl.pallas_call(
        _gelu_body,
        grid=(B // 1024,),
        in_specs=[_row_spec()],
        out_specs=_row_spec(),
        out_shape=jax.ShapeDtypeStruct((B, D), _f32),
    )(x)


def kernel(x, edge_index, idx, edge_type, edge_weight, Wp, bp, bn_gamma, bn_beta,
           W_input0, b_input0, W_rel0, W_out0, b_out0,
           W_input1, b_input1, W_rel1, W_out1, b_out1):
    row = edge_index[0]
    col = edge_index[1]
    pad = EPAD - E
    rowp = jnp.concatenate([row, jnp.full((pad,), DUMMY, _i32)])
    colp = jnp.concatenate([col, jnp.full((pad,), DUMMY, _i32)])
    relp = jnp.concatenate([edge_type, jnp.zeros((pad,), _i32)])
    xpad = jnp.pad(x, ((0, NPAD - N), (0, 0)))

    bp2 = bp.reshape(1, D)
    g2 = bn_gamma.reshape(1, D)
    bt2 = bn_beta.reshape(1, D)

    deg = _deg_kernel(colp)

    stats = _tc_stats(xpad, Wp, bp2)
    h = _tc_h(xpad, Wp, bp2, g2, bt2, stats)

    for (Wi, bi, Wr, Wo, bo) in (
            (W_input0, b_input0, W_rel0, W_out0, b_out0),
            (W_input1, b_input1, W_rel1, W_out1, b_out1)):
        xs, ts = _tc_prep(h, deg, Wi, bi.reshape(1, D), Wr.reshape(R * D, D))
        sc = _scan_kernel(rowp, colp, relp, *xs, *ts)
        h = _tc_finish(sc, deg, Wo, bo.reshape(1, D))

    hb = _take_kernel(h, idx)
    return _tc_gelu(hb)
